# baseline (speedup 1.0000x reference)
import jax
import jax.numpy as jnp
import numpy as np
from jax import lax
from jax.experimental import pallas as pl
from jax.experimental.pallas import tpu as pltpu
from jax.experimental.pallas import tpu_sc as plsc

N = 10000
E = 320000
C = 4
D = 3
F = 128
ED = 16
EIN = 2 * F + C * C + ED
EH = 2 * EIN
HID = 128
NH = 2 * (HID + F)
CD16 = 16

NC = 2
NS = 16
NW = NC * NS
EPW = E // NW
KG = 80
NCH = EPW // KG

BE = 1280
BN = 1000
NP = 10240
NP8 = NP // 8
TW = 256
XK = 2 * F + 128 + ED


def _silu(x):
    return x * jax.nn.sigmoid(x)


def _dot(a, b):
    return jnp.dot(a, b, preferred_element_type=jnp.float32)


def _dotb(a, b):
    return jnp.dot(a.astype(jnp.bfloat16), b.astype(jnp.bfloat16),
                   preferred_element_type=jnp.float32)


def _gather_body(tab_hbm, row_hbm, col_hbm,
                 rd_out, cd_out,
                 ridx, cidx, dbr, dbc, sem):
    cid = lax.axis_index("c")
    sid = lax.axis_index("s")
    wid = sid * NC + cid
    base0 = wid * EPW

    def body(t, carry):
        base = pl.multiple_of(base0 + t * KG, 8)
        pltpu.sync_copy(row_hbm.at[pl.ds(base, KG)], ridx)
        pltpu.sync_copy(col_hbm.at[pl.ds(base, KG)], cidx)
        d1 = pltpu.async_copy(tab_hbm.at[ridx], dbr, sem)
        d2 = pltpu.async_copy(tab_hbm.at[cidx], dbc, sem)
        d1.wait()
        d2.wait()
        pltpu.sync_copy(dbr, rd_out.at[pl.ds(base, KG)])
        pltpu.sync_copy(dbc, cd_out.at[pl.ds(base, KG)])
        return carry

    lax.fori_loop(0, NCH, body, 0)


def _scatter_body(ef_hbm, tr_hbm, row_hbm, row8_hbm, zh_hbm, zt_hbm,
                  aggh_out, aggc_out,
                  sh_h, sh_t, efb, trb, idxb, i8b, sem):
    cid = lax.axis_index("c")
    sid = lax.axis_index("s")
    wid = sid * NC + cid
    base0 = wid * EPW
    ipt = NP // (NS * KG)

    def ibody(t, carry):
        off = pl.multiple_of((sid * ipt + t) * KG, 8)
        pltpu.sync_copy(zh_hbm.at[pl.ds(off, KG)], efb)
        pltpu.sync_copy(efb, sh_h.at[pl.ds(off, KG)])
        return carry

    lax.fori_loop(0, ipt, ibody, 0)
    off8 = pl.multiple_of(sid * KG, 8)
    pltpu.sync_copy(zt_hbm.at[pl.ds(off8, KG)], trb)
    pltpu.sync_copy(trb, sh_t.at[pl.ds(off8, KG)])
    plsc.subcore_barrier()

    def body(t, carry):
        base = pl.multiple_of(base0 + t * KG, 8)
        pltpu.sync_copy(row_hbm.at[pl.ds(base, KG)], idxb)
        pltpu.sync_copy(row8_hbm.at[pl.ds(base, KG)], i8b)
        d1 = pltpu.async_copy(ef_hbm.at[pl.ds(base, KG)], efb, sem)
        d2 = pltpu.async_copy(tr_hbm.at[pl.ds(base, KG)], trb, sem)
        d1.wait()
        d2.wait()
        pltpu.sync_copy(efb, sh_h.at[idxb], add=True)
        pltpu.sync_copy(trb, sh_t.at[i8b], add=True)
        return carry

    lax.fori_loop(0, NCH, body, 0)
    plsc.subcore_barrier()

    def obody(t, carry):
        off = pl.multiple_of((sid * ipt + t) * KG, 8)
        dst = pl.multiple_of(cid * NP + (sid * ipt + t) * KG, 8)
        pltpu.sync_copy(sh_h.at[pl.ds(off, KG)], efb)
        pltpu.sync_copy(efb, aggh_out.at[pl.ds(dst, KG)])
        return carry

    lax.fori_loop(0, ipt, obody, 0)
    dst8 = pl.multiple_of(cid * NP8 + sid * KG, 8)
    pltpu.sync_copy(sh_t.at[pl.ds(off8, KG)], trb)
    pltpu.sync_copy(trb, aggc_out.at[pl.ds(dst8, KG)])


def _edge_body(rd, cdat, ea, oh, Wfull, be1, We2, be2,
               S1, S2, Wc1, Wc2e, cnt_row, Rrep, ef_out, tr_out):
    rp = rd[...]
    cp = cdat[...]
    cd = (rp[:, F:F + CD16].astype(jnp.float32)
          - cp[:, F:F + CD16].astype(jnp.float32))
    p = _dotb(cd, S1[...]) * _dotb(cd, S2[...])
    x = jnp.concatenate(
        [rp[:, :F], cp[:, :F], p.astype(jnp.bfloat16),
         ea[...].astype(jnp.bfloat16)], axis=1)
    t = _silu(_dot(x, Wfull[...]) + be1[...])
    ef = _silu(_dotb(t, We2[...]) + be2[...])
    cw = _silu(_dotb(ef, Wc1[...]))
    cwe = _dotb(cw, Wc2e[...])
    ef_out[...] = ef
    tr16 = cd * cwe + cnt_row[...]
    tr_out[...] = _dotb(oh[...], Rrep[...]) * jnp.concatenate([tr16] * 8, 1)


def _node_body(h, a0, a1, c0, c1, co16, Wnh, Wna, bn1, Wn2, bn2, E15,
               ho_out, co_out):
    ah = a0[...] + a1[...]
    t = _silu(_dotb(h[...], Wnh[...]) + _dotb(ah, Wna[...]) + bn1[...])
    ho_out[...] = h[...] + _silu(_dotb(t, Wn2[...]) + bn2[...])
    ac = c0[...] + c1[...]
    deg = _dot(ac, E15[...])
    agg = ac / jnp.maximum(deg, 1.0)
    co_out[...] = co16[...] + jnp.clip(agg, -10.0, 10.0)


def _full(shape):
    nd = len(shape)
    return pl.BlockSpec(shape, lambda i, _nd=nd: (0,) * _nd)


def _rows(block, width):
    return pl.BlockSpec((block, width), lambda i: (i, 0))


def kernel(h, edge_index, coord, edge_attr, We1, be1, We2, be2,
           Wn1, bn1, Wn2, bn2, Wc1, Wc2):
    row = edge_index[0]
    col = edge_index[1]
    coord16 = jnp.pad(coord.reshape(N, C * D), ((0, 0), (0, CD16 - C * D)))

    s1 = np.zeros((CD16, 128), np.float32)
    s2 = np.zeros((CD16, 128), np.float32)
    perm = np.zeros(C * C * D, np.int64)
    for i in range(C):
        for j in range(C):
            for d in range(D):
                k = 12 * i + 3 * j + d
                s1[3 * i + d, k] = 1.0
                s2[3 * j + d, k] = 1.0
                perm[k] = 2 * F + C * i + j
    S1 = jnp.asarray(s1)
    S2 = jnp.asarray(s2)
    Wr = jnp.zeros((128, EH), jnp.float32).at[:C * C * D].set(We1[perm])
    Wa = We1[:F]
    Wb = We1[F:2 * F]
    We = We1[2 * F + C * C:]
    cmap = np.repeat(np.arange(C), D)
    Wc2e = jnp.concatenate(
        [Wc2[:, cmap], jnp.zeros((2 * HID, CD16 - C * D), jnp.float32)], axis=1)
    cnt_row = jnp.zeros((1, CD16), jnp.float32).at[0, CD16 - 1].set(1.0)
    rrep = np.zeros((8, 128), np.float32)
    for k in range(8):
        rrep[k, 16 * k:16 * (k + 1)] = 1.0
    Rrep = jnp.asarray(rrep)
    e15 = np.zeros((CD16, CD16), np.float32)
    e15[CD16 - 1, :] = 1.0
    E15 = jnp.asarray(e15)
    Wnh = Wn1[:F]
    Wna = Wn1[F:]
    be1_2 = be1.reshape(1, EH)
    be2_2 = be2.reshape(1, HID)
    bn1_2 = bn1.reshape(1, NH)
    bn2_2 = bn2.reshape(1, F)
    zh = jnp.zeros((NP, HID), jnp.float32)
    zt = jnp.zeros((NP8, 128), jnp.float32)
    oh = jax.nn.one_hot(row % 8, 8, dtype=jnp.float32)
    row8 = row // 8

    mesh = plsc.VectorSubcoreMesh(core_axis_name="c", subcore_axis_name="s")

    table_bf = jnp.concatenate(
        [h.astype(jnp.bfloat16), coord16.astype(jnp.bfloat16),
         jnp.zeros((N, TW - F - CD16), jnp.bfloat16)], axis=1)
    table = lax.bitcast_convert_type(
        table_bf.reshape(N, TW // 2, 2), jnp.uint32)
    Wfull = jnp.concatenate([Wa, Wb, Wr, We], axis=0).astype(jnp.bfloat16)

    gather = pl.kernel(
        _gather_body,
        out_type=[jax.ShapeDtypeStruct((E, TW // 2), jnp.uint32),
                  jax.ShapeDtypeStruct((E, TW // 2), jnp.uint32)],
        mesh=mesh,
        scratch_types=[pltpu.VMEM((KG,), jnp.int32),
                       pltpu.VMEM((KG,), jnp.int32),
                       pltpu.VMEM((KG, TW // 2), jnp.uint32),
                       pltpu.VMEM((KG, TW // 2), jnp.uint32),
                       pltpu.SemaphoreType.DMA],
    )
    rdat_u, cdat_u = gather(table, row, col)
    rdat = lax.bitcast_convert_type(rdat_u, jnp.bfloat16).reshape(E, TW)
    cdat = lax.bitcast_convert_type(cdat_u, jnp.bfloat16).reshape(E, TW)

    ef, tr128 = pl.pallas_call(
        _edge_body,
        grid=(E // BE,),
        in_specs=[_rows(BE, TW), _rows(BE, TW), _rows(BE, ED), _rows(BE, 8),
                  _full((XK, EH)), _full((1, EH)), _full((EH, HID)),
                  _full((1, HID)), _full((CD16, 128)), _full((CD16, 128)),
                  _full((HID, 2 * HID)), _full((2 * HID, CD16)),
                  _full((1, CD16)), _full((8, 128))],
        out_specs=[_rows(BE, HID), _rows(BE, 128)],
        out_shape=[jax.ShapeDtypeStruct((E, HID), jnp.float32),
                   jax.ShapeDtypeStruct((E, 128), jnp.float32)],
    )(rdat, cdat, edge_attr, oh, Wfull, be1_2, We2, be2_2,
      S1, S2, Wc1, Wc2e, cnt_row, Rrep)

    scatter = pl.kernel(
        _scatter_body,
        out_type=[jax.ShapeDtypeStruct((NC * NP, HID), jnp.float32),
                  jax.ShapeDtypeStruct((NC * NP8, 128), jnp.float32)],
        mesh=mesh,
        scratch_types=[pltpu.VMEM_SHARED((NP, HID), jnp.float32),
                       pltpu.VMEM_SHARED((NP8, 128), jnp.float32),
                       pltpu.VMEM((KG, HID), jnp.float32),
                       pltpu.VMEM((KG, 128), jnp.float32),
                       pltpu.VMEM((KG,), jnp.int32),
                       pltpu.VMEM((KG,), jnp.int32),
                       pltpu.SemaphoreType.DMA],
    )
    aggh_p, aggc_p = scatter(ef, tr128, row, row8, zh, zt)
    aggc16 = aggc_p.reshape(NC * NP, CD16)

    h_out, co16 = pl.pallas_call(
        _node_body,
        grid=(N // BN,),
        in_specs=[_rows(BN, F), _rows(BN, HID), _rows(BN, HID),
                  _rows(BN, CD16), _rows(BN, CD16), _rows(BN, CD16),
                  _full((F, NH)), _full((HID, NH)), _full((1, NH)),
                  _full((NH, F)), _full((1, F)), _full((CD16, CD16))],
        out_specs=[_rows(BN, F), _rows(BN, CD16)],
        out_shape=[jax.ShapeDtypeStruct((N, F), jnp.float32),
                   jax.ShapeDtypeStruct((N, CD16), jnp.float32)],
    )(h, aggh_p[:N], aggh_p[NP:NP + N], aggc16[:N], aggc16[NP:NP + N],
      coord16, Wnh, Wna, bn1_2, Wn2, bn2_2, E15)

    coord_out = co16[:, :C * D].reshape(N, C, D)
    return (h_out, coord_out)


# speedup vs baseline: 2.2673x; 2.2673x over previous
import jax
import jax.numpy as jnp
import numpy as np
from jax import lax
from jax.experimental import pallas as pl
from jax.experimental.pallas import tpu as pltpu
from jax.experimental.pallas import tpu_sc as plsc

N = 10000
E = 320000
C = 4
D = 3
F = 128
ED = 16
EIN = 2 * F + C * C + ED
EH = 2 * EIN
HID = 128
NH = 2 * (HID + F)
CD16 = 16

NC = 2
NS = 16
NW = NC * NS
EPW = E // NW
KG = 80
NCH = EPW // KG

BE = 1280
BN = 1000
NP = 10240
NP8 = NP // 8
TW = 256
XK = 2 * F + 128 + ED


def _silu(x):
    return x * jax.nn.sigmoid(x)


def _dot(a, b):
    return jnp.dot(a, b, preferred_element_type=jnp.float32)


def _dotb(a, b):
    return jnp.dot(a.astype(jnp.bfloat16), b.astype(jnp.bfloat16),
                   preferred_element_type=jnp.float32)


def _gather_body(tab_hbm, row_hbm, col_hbm,
                 rd_out, cd_out,
                 ridx, cidx, dbr, dbc, sem):
    cid = lax.axis_index("c")
    sid = lax.axis_index("s")
    wid = sid * NC + cid
    base0 = wid * EPW

    def body(t, carry):
        base = pl.multiple_of(base0 + t * KG, 8)
        pltpu.sync_copy(row_hbm.at[pl.ds(base, KG)], ridx)
        pltpu.sync_copy(col_hbm.at[pl.ds(base, KG)], cidx)
        d1 = pltpu.async_copy(tab_hbm.at[ridx], dbr, sem)
        d2 = pltpu.async_copy(tab_hbm.at[cidx], dbc, sem)
        d1.wait()
        d2.wait()
        pltpu.sync_copy(dbr, rd_out.at[pl.ds(base, KG)])
        pltpu.sync_copy(dbc, cd_out.at[pl.ds(base, KG)])
        return carry

    lax.fori_loop(0, NCH, body, 0)


def _scatter_body(ef_hbm, tr_hbm, row_hbm, row8_hbm, zh_hbm, zt_hbm,
                  aggh_out, aggc_out,
                  sh_h, sh_t, efb, trb, idxb, i8b, sem):
    cid = lax.axis_index("c")
    sid = lax.axis_index("s")
    wid = sid * NC + cid
    base0 = wid * EPW
    ipt = NP // (NS * KG)

    def ibody(t, carry):
        off = pl.multiple_of((sid * ipt + t) * KG, 8)
        pltpu.sync_copy(zh_hbm.at[pl.ds(off, KG)], efb)
        pltpu.sync_copy(efb, sh_h.at[pl.ds(off, KG)])
        return carry

    lax.fori_loop(0, ipt, ibody, 0)
    off8 = pl.multiple_of(sid * KG, 8)
    pltpu.sync_copy(zt_hbm.at[pl.ds(off8, KG)], trb)
    pltpu.sync_copy(trb, sh_t.at[pl.ds(off8, KG)])
    plsc.subcore_barrier()

    def body(t, carry):
        base = pl.multiple_of(base0 + t * KG, 8)
        pltpu.sync_copy(row_hbm.at[pl.ds(base, KG)], idxb)
        pltpu.sync_copy(row8_hbm.at[pl.ds(base, KG)], i8b)
        d1 = pltpu.async_copy(ef_hbm.at[pl.ds(base, KG)], efb, sem)
        d2 = pltpu.async_copy(tr_hbm.at[pl.ds(base, KG)], trb, sem)
        d1.wait()
        d2.wait()
        pltpu.sync_copy(efb, sh_h.at[idxb], add=True)
        pltpu.sync_copy(trb, sh_t.at[i8b], add=True)
        return carry

    lax.fori_loop(0, NCH, body, 0)
    plsc.subcore_barrier()

    def obody(t, carry):
        off = pl.multiple_of((sid * ipt + t) * KG, 8)
        dst = pl.multiple_of(cid * NP + (sid * ipt + t) * KG, 8)
        pltpu.sync_copy(sh_h.at[pl.ds(off, KG)], efb)
        pltpu.sync_copy(efb, aggh_out.at[pl.ds(dst, KG)])
        return carry

    lax.fori_loop(0, ipt, obody, 0)
    dst8 = pl.multiple_of(cid * NP8 + sid * KG, 8)
    pltpu.sync_copy(sh_t.at[pl.ds(off8, KG)], trb)
    pltpu.sync_copy(trb, aggc_out.at[pl.ds(dst8, KG)])


def _blo(x):
    return lax.bitcast_convert_type(x << 16, jnp.float32)


def _bhi(x):
    return lax.bitcast_convert_type(x & jnp.uint32(0xFFFF0000), jnp.float32)


def _edge_body(rd, cdat, ea, oh, Wfull, be1, We2, be2,
               S1, S2, Wc1, Wc2e, cnt_row, Rrep, ef_out, tr_out):
    rp = rd[...]
    cp = cdat[...]
    cdl = _blo(rp[:, 64:72]) - _blo(cp[:, 64:72])
    cdh = _bhi(rp[:, 64:72]) - _bhi(cp[:, 64:72])
    cd = jnp.concatenate([cdl, cdh], axis=1)
    p = _dotb(cd, S1[...]) * _dotb(cd, S2[...])
    x = jnp.concatenate(
        [_blo(rp[:, :64]), _bhi(rp[:, :64]),
         _blo(cp[:, :64]), _bhi(cp[:, :64]),
         p, ea[...]], axis=1).astype(jnp.bfloat16)
    t = _silu(_dot(x, Wfull[...]) + be1[...])
    ef = _silu(_dotb(t, We2[...]) + be2[...])
    cw = _silu(_dotb(ef, Wc1[...]))
    cwe = _dotb(cw, Wc2e[...])
    ef_out[...] = ef
    tr16 = cd * cwe + cnt_row[...]
    tr_out[...] = _dotb(oh[...], Rrep[...]) * jnp.concatenate([tr16] * 8, 1)


def _node_body(h, a0, a1, c0, c1, co16, Wnh, Wna, bn1, Wn2, bn2, E15,
               ho_out, co_out):
    ah = a0[...] + a1[...]
    t = _silu(_dotb(h[...], Wnh[...]) + _dotb(ah, Wna[...]) + bn1[...])
    ho_out[...] = h[...] + _silu(_dotb(t, Wn2[...]) + bn2[...])
    ac = c0[...] + c1[...]
    deg = _dot(ac, E15[...])
    agg = ac / jnp.maximum(deg, 1.0)
    co_out[...] = co16[...] + jnp.clip(agg, -10.0, 10.0)


def _full(shape):
    nd = len(shape)
    return pl.BlockSpec(shape, lambda i, _nd=nd: (0,) * _nd)


def _rows(block, width):
    return pl.BlockSpec((block, width), lambda i: (i, 0))


def kernel(h, edge_index, coord, edge_attr, We1, be1, We2, be2,
           Wn1, bn1, Wn2, bn2, Wc1, Wc2):
    row = edge_index[0]
    col = edge_index[1]
    coord16 = jnp.pad(coord.reshape(N, C * D), ((0, 0), (0, CD16 - C * D)))

    s1 = np.zeros((CD16, 128), np.float32)
    s2 = np.zeros((CD16, 128), np.float32)
    perm = np.zeros(C * C * D, np.int64)
    for i in range(C):
        for j in range(C):
            for d in range(D):
                k = 12 * i + 3 * j + d
                s1[3 * i + d, k] = 1.0
                s2[3 * j + d, k] = 1.0
                perm[k] = 2 * F + C * i + j
    S1 = jnp.asarray(s1)
    S2 = jnp.asarray(s2)
    Wr = jnp.zeros((128, EH), jnp.float32).at[:C * C * D].set(We1[perm])
    Wa = We1[:F]
    Wb = We1[F:2 * F]
    We = We1[2 * F + C * C:]
    cmap = np.repeat(np.arange(C), D)
    Wc2e = jnp.concatenate(
        [Wc2[:, cmap], jnp.zeros((2 * HID, CD16 - C * D), jnp.float32)], axis=1)
    cnt_row = jnp.zeros((1, CD16), jnp.float32).at[0, CD16 - 1].set(1.0)
    rrep = np.zeros((8, 128), np.float32)
    for k in range(8):
        rrep[k, 16 * k:16 * (k + 1)] = 1.0
    Rrep = jnp.asarray(rrep)
    e15 = np.zeros((CD16, CD16), np.float32)
    e15[CD16 - 1, :] = 1.0
    E15 = jnp.asarray(e15)
    Wnh = Wn1[:F]
    Wna = Wn1[F:]
    be1_2 = be1.reshape(1, EH)
    be2_2 = be2.reshape(1, HID)
    bn1_2 = bn1.reshape(1, NH)
    bn2_2 = bn2.reshape(1, F)
    zh = jnp.zeros((NP, HID), jnp.float32)
    zt = jnp.zeros((NP8, 128), jnp.float32)
    oh = jax.nn.one_hot(row % 8, 8, dtype=jnp.float32)
    row8 = row // 8

    mesh = plsc.VectorSubcoreMesh(core_axis_name="c", subcore_axis_name="s")

    hu = lax.bitcast_convert_type(
        h.astype(jnp.bfloat16), jnp.uint16).astype(jnp.uint32)
    cu = lax.bitcast_convert_type(
        coord16.astype(jnp.bfloat16), jnp.uint16).astype(jnp.uint32)
    table = jnp.concatenate(
        [hu[:, :64] | (hu[:, 64:] << 16),
         cu[:, :8] | (cu[:, 8:] << 16),
         jnp.zeros((N, TW // 2 - 72), jnp.uint32)], axis=1)
    Wfull = jnp.concatenate([Wa, Wb, Wr, We], axis=0).astype(jnp.bfloat16)

    gather = pl.kernel(
        _gather_body,
        out_type=[jax.ShapeDtypeStruct((E, TW // 2), jnp.uint32),
                  jax.ShapeDtypeStruct((E, TW // 2), jnp.uint32)],
        mesh=mesh,
        scratch_types=[pltpu.VMEM((KG,), jnp.int32),
                       pltpu.VMEM((KG,), jnp.int32),
                       pltpu.VMEM((KG, TW // 2), jnp.uint32),
                       pltpu.VMEM((KG, TW // 2), jnp.uint32),
                       pltpu.SemaphoreType.DMA],
    )
    rdat, cdat = gather(table, row, col)

    ef, tr128 = pl.pallas_call(
        _edge_body,
        grid=(E // BE,),
        in_specs=[_rows(BE, TW // 2), _rows(BE, TW // 2), _rows(BE, ED),
                  _rows(BE, 8),
                  _full((XK, EH)), _full((1, EH)), _full((EH, HID)),
                  _full((1, HID)), _full((CD16, 128)), _full((CD16, 128)),
                  _full((HID, 2 * HID)), _full((2 * HID, CD16)),
                  _full((1, CD16)), _full((8, 128))],
        out_specs=[_rows(BE, HID), _rows(BE, 128)],
        out_shape=[jax.ShapeDtypeStruct((E, HID), jnp.float32),
                   jax.ShapeDtypeStruct((E, 128), jnp.float32)],
    )(rdat, cdat, edge_attr, oh, Wfull, be1_2, We2, be2_2,
      S1, S2, Wc1, Wc2e, cnt_row, Rrep)

    scatter = pl.kernel(
        _scatter_body,
        out_type=[jax.ShapeDtypeStruct((NC * NP, HID), jnp.float32),
                  jax.ShapeDtypeStruct((NC * NP8, 128), jnp.float32)],
        mesh=mesh,
        scratch_types=[pltpu.VMEM_SHARED((NP, HID), jnp.float32),
                       pltpu.VMEM_SHARED((NP8, 128), jnp.float32),
                       pltpu.VMEM((KG, HID), jnp.float32),
                       pltpu.VMEM((KG, 128), jnp.float32),
                       pltpu.VMEM((KG,), jnp.int32),
                       pltpu.VMEM((KG,), jnp.int32),
                       pltpu.SemaphoreType.DMA],
    )
    aggh_p, aggc_p = scatter(ef, tr128, row, row8, zh, zt)
    aggc16 = aggc_p.reshape(NC * NP, CD16)

    h_out, co16 = pl.pallas_call(
        _node_body,
        grid=(N // BN,),
        in_specs=[_rows(BN, F), _rows(BN, HID), _rows(BN, HID),
                  _rows(BN, CD16), _rows(BN, CD16), _rows(BN, CD16),
                  _full((F, NH)), _full((HID, NH)), _full((1, NH)),
                  _full((NH, F)), _full((1, F)), _full((CD16, CD16))],
        out_specs=[_rows(BN, F), _rows(BN, CD16)],
        out_shape=[jax.ShapeDtypeStruct((N, F), jnp.float32),
                   jax.ShapeDtypeStruct((N, CD16), jnp.float32)],
    )(h, aggh_p[:N], aggh_p[NP:NP + N], aggc16[:N], aggc16[NP:NP + N],
      coord16, Wnh, Wna, bn1_2, Wn2, bn2_2, E15)

    coord_out = co16[:, :C * D].reshape(N, C, D)
    return (h_out, coord_out)


# speedup vs baseline: 2.2694x; 1.0010x over previous
import jax
import jax.numpy as jnp
import numpy as np
from jax import lax
from jax.experimental import pallas as pl
from jax.experimental.pallas import tpu as pltpu
from jax.experimental.pallas import tpu_sc as plsc

N = 10000
E = 320000
C = 4
D = 3
F = 128
ED = 16
EIN = 2 * F + C * C + ED
EH = 2 * EIN
HID = 128
NH = 2 * (HID + F)
CD16 = 16

NC = 2
NS = 16
NW = NC * NS
EPW = E // NW
KG = 80
NCH = EPW // KG

BE = 1280
BN = 1000
NP = 10240
NP8 = NP // 8
TW = 256
XK = 2 * F + 128 + ED


def _silu(x):
    return x * jax.nn.sigmoid(x)


def _dot(a, b):
    return jnp.dot(a, b, preferred_element_type=jnp.float32)


def _dotb(a, b):
    return jnp.dot(a.astype(jnp.bfloat16), b.astype(jnp.bfloat16),
                   preferred_element_type=jnp.float32)


def _gather_body(tab_hbm, row_hbm, col_hbm,
                 rd_out, cd_out,
                 ridx, cidx, dbr, dbc, sem):
    cid = lax.axis_index("c")
    sid = lax.axis_index("s")
    wid = sid * NC + cid
    base0 = wid * EPW

    def body(t, carry):
        base = pl.multiple_of(base0 + t * KG, 8)
        pltpu.sync_copy(row_hbm.at[pl.ds(base, KG)], ridx)
        pltpu.sync_copy(col_hbm.at[pl.ds(base, KG)], cidx)
        d1 = pltpu.async_copy(tab_hbm.at[ridx], dbr, sem)
        d2 = pltpu.async_copy(tab_hbm.at[cidx], dbc, sem)
        d1.wait()
        d2.wait()
        pltpu.sync_copy(dbr, rd_out.at[pl.ds(base, KG)])
        pltpu.sync_copy(dbc, cd_out.at[pl.ds(base, KG)])
        return carry

    lax.fori_loop(0, NCH, body, 0)


def _scatter_body(ef_hbm, tr_hbm, row_hbm, row8_hbm, zh_hbm, zt_hbm,
                  aggh_out, aggc_out,
                  sh_h, sh_t, efb, trb, idxb, i8b, sem):
    cid = lax.axis_index("c")
    sid = lax.axis_index("s")
    wid = sid * NC + cid
    base0 = wid * EPW
    ipt = NP // (NS * KG)

    def ibody(t, carry):
        off = pl.multiple_of((sid * ipt + t) * KG, 8)
        pltpu.sync_copy(zh_hbm.at[pl.ds(off, KG)], efb)
        pltpu.sync_copy(efb, sh_h.at[pl.ds(off, KG)])
        return carry

    lax.fori_loop(0, ipt, ibody, 0)
    off8 = pl.multiple_of(sid * KG, 8)
    pltpu.sync_copy(zt_hbm.at[pl.ds(off8, KG)], trb)
    pltpu.sync_copy(trb, sh_t.at[pl.ds(off8, KG)])
    plsc.subcore_barrier()

    def body(t, carry):
        base = pl.multiple_of(base0 + t * KG, 8)
        pltpu.sync_copy(row_hbm.at[pl.ds(base, KG)], idxb)
        pltpu.sync_copy(row8_hbm.at[pl.ds(base, KG)], i8b)
        d1 = pltpu.async_copy(ef_hbm.at[pl.ds(base, KG)], efb, sem)
        d2 = pltpu.async_copy(tr_hbm.at[pl.ds(base, KG)], trb, sem)
        d1.wait()
        d2.wait()
        pltpu.sync_copy(efb, sh_h.at[idxb], add=True)
        pltpu.sync_copy(trb, sh_t.at[i8b], add=True)
        return carry

    lax.fori_loop(0, NCH, body, 0)
    plsc.subcore_barrier()

    def obody(t, carry):
        off = pl.multiple_of((sid * ipt + t) * KG, 8)
        dst = pl.multiple_of(cid * NP + (sid * ipt + t) * KG, 8)
        pltpu.sync_copy(sh_h.at[pl.ds(off, KG)], efb)
        pltpu.sync_copy(efb, aggh_out.at[pl.ds(dst, KG)])
        return carry

    lax.fori_loop(0, ipt, obody, 0)
    dst8 = pl.multiple_of(cid * NP8 + sid * KG, 8)
    pltpu.sync_copy(sh_t.at[pl.ds(off8, KG)], trb)
    pltpu.sync_copy(trb, aggc_out.at[pl.ds(dst8, KG)])


def _blo(x):
    return lax.bitcast_convert_type(x << 16, jnp.float32)


def _bhi(x):
    return lax.bitcast_convert_type(x & jnp.uint32(0xFFFF0000), jnp.float32)


def _edge_body(rd, cdat, ea, Wfull, be1, We2, be2,
               S1, S2, Wc1, Wc2e, cnt_row, Rrep, ef_out, tr_out):
    rp = rd[...]
    cp = cdat[...]
    cdl = _blo(rp[:, 64:72]) - _blo(cp[:, 64:72])
    cdh = _bhi(rp[:, 64:72]) - _bhi(cp[:, 64:72])
    cd = jnp.concatenate([cdl, cdh], axis=1)
    oh = jnp.concatenate(
        [_blo(rp[:, 72:76]), _bhi(rp[:, 72:76])], axis=1)
    p = _dotb(cd, S1[...]) * _dotb(cd, S2[...])
    x = jnp.concatenate(
        [_blo(rp[:, :64]), _bhi(rp[:, :64]),
         _blo(cp[:, :64]), _bhi(cp[:, :64]),
         p, ea[...]], axis=1).astype(jnp.bfloat16)
    t = _silu(_dot(x, Wfull[...]) + be1[...])
    ef = _silu(_dotb(t, We2[...]) + be2[...])
    cw = _silu(_dotb(ef, Wc1[...]))
    cwe = _dotb(cw, Wc2e[...])
    ef_out[...] = ef
    tr16 = cd * cwe + cnt_row[...]
    tr_out[...] = _dotb(oh, Rrep[...]) * jnp.concatenate([tr16] * 8, 1)


def _node_body(h, a0, a1, c0, c1, co16, Wnh, Wna, bn1, Wn2, bn2, E15,
               ho_out, co_out):
    ah = a0[...] + a1[...]
    t = _silu(_dotb(h[...], Wnh[...]) + _dotb(ah, Wna[...]) + bn1[...])
    ho_out[...] = h[...] + _silu(_dotb(t, Wn2[...]) + bn2[...])
    ac = c0[...] + c1[...]
    deg = _dot(ac, E15[...])
    agg = ac / jnp.maximum(deg, 1.0)
    co_out[...] = co16[...] + jnp.clip(agg, -10.0, 10.0)


def _full(shape):
    nd = len(shape)
    return pl.BlockSpec(shape, lambda i, _nd=nd: (0,) * _nd)


def _rows(block, width):
    return pl.BlockSpec((block, width), lambda i: (i, 0))


def kernel(h, edge_index, coord, edge_attr, We1, be1, We2, be2,
           Wn1, bn1, Wn2, bn2, Wc1, Wc2):
    row = edge_index[0]
    col = edge_index[1]
    coord16 = jnp.pad(coord.reshape(N, C * D), ((0, 0), (0, CD16 - C * D)))

    s1 = np.zeros((CD16, 128), np.float32)
    s2 = np.zeros((CD16, 128), np.float32)
    perm = np.zeros(C * C * D, np.int64)
    for i in range(C):
        for j in range(C):
            for d in range(D):
                k = 12 * i + 3 * j + d
                s1[3 * i + d, k] = 1.0
                s2[3 * j + d, k] = 1.0
                perm[k] = 2 * F + C * i + j
    S1 = jnp.asarray(s1)
    S2 = jnp.asarray(s2)
    Wr = jnp.zeros((128, EH), jnp.float32).at[:C * C * D].set(We1[perm])
    Wa = We1[:F]
    Wb = We1[F:2 * F]
    We = We1[2 * F + C * C:]
    cmap = np.repeat(np.arange(C), D)
    Wc2e = jnp.concatenate(
        [Wc2[:, cmap], jnp.zeros((2 * HID, CD16 - C * D), jnp.float32)], axis=1)
    cnt_row = jnp.zeros((1, CD16), jnp.float32).at[0, CD16 - 1].set(1.0)
    rrep = np.zeros((8, 128), np.float32)
    for k in range(8):
        rrep[k, 16 * k:16 * (k + 1)] = 1.0
    Rrep = jnp.asarray(rrep)
    e15 = np.zeros((CD16, CD16), np.float32)
    e15[CD16 - 1, :] = 1.0
    E15 = jnp.asarray(e15)
    Wnh = Wn1[:F]
    Wna = Wn1[F:]
    be1_2 = be1.reshape(1, EH)
    be2_2 = be2.reshape(1, HID)
    bn1_2 = bn1.reshape(1, NH)
    bn2_2 = bn2.reshape(1, F)
    zh = jnp.zeros((NP, HID), jnp.float32)
    zt = jnp.zeros((NP8, 128), jnp.float32)
    row8 = row // 8

    mesh = plsc.VectorSubcoreMesh(core_axis_name="c", subcore_axis_name="s")

    hu = lax.bitcast_convert_type(
        h.astype(jnp.bfloat16), jnp.uint16).astype(jnp.uint32)
    cu = lax.bitcast_convert_type(
        coord16.astype(jnp.bfloat16), jnp.uint16).astype(jnp.uint32)
    ou = lax.bitcast_convert_type(
        jax.nn.one_hot(jnp.arange(N) % 8, 8, dtype=jnp.bfloat16),
        jnp.uint16).astype(jnp.uint32)
    table = jnp.concatenate(
        [hu[:, :64] | (hu[:, 64:] << 16),
         cu[:, :8] | (cu[:, 8:] << 16),
         ou[:, :4] | (ou[:, 4:] << 16),
         jnp.zeros((N, TW // 2 - 76), jnp.uint32)], axis=1)
    Wfull = jnp.concatenate([Wa, Wb, Wr, We], axis=0).astype(jnp.bfloat16)

    gather = pl.kernel(
        _gather_body,
        out_type=[jax.ShapeDtypeStruct((E, TW // 2), jnp.uint32),
                  jax.ShapeDtypeStruct((E, TW // 2), jnp.uint32)],
        mesh=mesh,
        scratch_types=[pltpu.VMEM((KG,), jnp.int32),
                       pltpu.VMEM((KG,), jnp.int32),
                       pltpu.VMEM((KG, TW // 2), jnp.uint32),
                       pltpu.VMEM((KG, TW // 2), jnp.uint32),
                       pltpu.SemaphoreType.DMA],
    )
    rdat, cdat = gather(table, row, col)

    ef, tr128 = pl.pallas_call(
        _edge_body,
        grid=(E // BE,),
        in_specs=[_rows(BE, TW // 2), _rows(BE, TW // 2), _rows(BE, ED),
                  _full((XK, EH)), _full((1, EH)), _full((EH, HID)),
                  _full((1, HID)), _full((CD16, 128)), _full((CD16, 128)),
                  _full((HID, 2 * HID)), _full((2 * HID, CD16)),
                  _full((1, CD16)), _full((8, 128))],
        out_specs=[_rows(BE, HID), _rows(BE, 128)],
        out_shape=[jax.ShapeDtypeStruct((E, HID), jnp.float32),
                   jax.ShapeDtypeStruct((E, 128), jnp.float32)],
    )(rdat, cdat, edge_attr, Wfull, be1_2, We2, be2_2,
      S1, S2, Wc1, Wc2e, cnt_row, Rrep)

    scatter = pl.kernel(
        _scatter_body,
        out_type=[jax.ShapeDtypeStruct((NC * NP, HID), jnp.float32),
                  jax.ShapeDtypeStruct((NC * NP8, 128), jnp.float32)],
        mesh=mesh,
        scratch_types=[pltpu.VMEM_SHARED((NP, HID), jnp.float32),
                       pltpu.VMEM_SHARED((NP8, 128), jnp.float32),
                       pltpu.VMEM((KG, HID), jnp.float32),
                       pltpu.VMEM((KG, 128), jnp.float32),
                       pltpu.VMEM((KG,), jnp.int32),
                       pltpu.VMEM((KG,), jnp.int32),
                       pltpu.SemaphoreType.DMA],
    )
    aggh_p, aggc_p = scatter(ef, tr128, row, row8, zh, zt)
    aggc16 = aggc_p.reshape(NC * NP, CD16)

    h_out, co16 = pl.pallas_call(
        _node_body,
        grid=(N // BN,),
        in_specs=[_rows(BN, F), _rows(BN, HID), _rows(BN, HID),
                  _rows(BN, CD16), _rows(BN, CD16), _rows(BN, CD16),
                  _full((F, NH)), _full((HID, NH)), _full((1, NH)),
                  _full((NH, F)), _full((1, F)), _full((CD16, CD16))],
        out_specs=[_rows(BN, F), _rows(BN, CD16)],
        out_shape=[jax.ShapeDtypeStruct((N, F), jnp.float32),
                   jax.ShapeDtypeStruct((N, CD16), jnp.float32)],
    )(h, aggh_p[:N], aggh_p[NP:NP + N], aggc16[:N], aggc16[NP:NP + N],
      coord16, Wnh, Wna, bn1_2, Wn2, bn2_2, E15)

    coord_out = co16[:, :C * D].reshape(N, C, D)
    return (h_out, coord_out)
